# baseline (speedup 1.0000x reference)
import functools

import jax
import jax.numpy as jnp
from jax import lax
from jax.experimental import pallas as pl
from jax.experimental.pallas import tpu as pltpu
from jax.experimental.pallas import tpu_sc as plsc

N = 10000
E = 320000
D = 128
H = 64
EPS_BN = 1e-5

NC = 2
NS = 16
NW = NC * NS
EPW = E // NW
K = 40
NCH = EPW // K
NG = NCH // 2
NPT = 632
NP = NS * NPT

NODE_BLK = 2000
EDGE_BLK = 6400


def _node_body(h_ref, aw, ab, bw, bb, dw, db, ew, eb,
               bew, beb, bem, bev,
               ah_o, db0_o, db1_o, eh_o, sce_o, she_o):
    x = h_ref[...]
    dn = (((1,), (1,)), ((), ()))
    ah = lax.dot_general(x, aw[...], dn, preferred_element_type=jnp.float32) + ab[...]
    bh = lax.dot_general(x, bw[...], dn, preferred_element_type=jnp.float32) + bb[...]
    dh = lax.dot_general(x, dw[...], dn, preferred_element_type=jnp.float32) + db[...]
    eh = lax.dot_general(x, ew[...], dn, preferred_element_type=jnp.float32) + eb[...]
    ah_o[...] = ah
    db0_o[...] = jnp.concatenate([dh[:, :H], bh[:, :H]], axis=1)
    db1_o[...] = jnp.concatenate([dh[:, H:], bh[:, H:]], axis=1)
    eh_o[...] = eh
    scale = bew[...] / jnp.sqrt(bev[...] + EPS_BN)
    sce_o[...] = scale
    she_o[...] = beb[...] - bem[...] * scale


def _node_tc(h, A_w, A_b, B_w, B_b, D_w, D_b, E_w, E_b,
             bn_e_w, bn_e_b, bn_e_mean, bn_e_var):
    grid = N // NODE_BLK
    row = pl.BlockSpec((NODE_BLK, D), lambda i: (i, 0))
    w = pl.BlockSpec((D, D), lambda i: (0, 0))
    v = pl.BlockSpec((1, D), lambda i: (0, 0))
    f32 = jnp.float32
    out_shape = [
        jax.ShapeDtypeStruct((N, D), f32),
        jax.ShapeDtypeStruct((N, D), f32),
        jax.ShapeDtypeStruct((N, D), f32),
        jax.ShapeDtypeStruct((N, D), f32),
        jax.ShapeDtypeStruct((1, D), f32),
        jax.ShapeDtypeStruct((1, D), f32),
    ]
    return pl.pallas_call(
        _node_body,
        grid=(grid,),
        in_specs=[row] + [w, v] * 4 + [v] * 4,
        out_specs=[row, row, row, row, v, v],
        out_shape=out_shape,
    )(h, A_w, A_b, B_w, B_b, D_w, D_b, E_w, E_b,
      bn_e_w, bn_e_b, bn_e_mean, bn_e_var)


def _ce_body(e_ref, cw, cb, ce0_o, ce1_o):
    dn = (((1,), (1,)), ((), ()))
    ce = lax.dot_general(e_ref[...], cw[...], dn,
                         preferred_element_type=jnp.float32) + cb[...]
    ce0_o[...] = ce[:, :H]
    ce1_o[...] = ce[:, H:]


def _ce_tc(e, C_w, C_b):
    grid = E // EDGE_BLK
    row = pl.BlockSpec((EDGE_BLK, D), lambda i: (i, 0))
    halfrow = pl.BlockSpec((EDGE_BLK, H), lambda i: (i, 0))
    f32 = jnp.float32
    return pl.pallas_call(
        _ce_body,
        grid=(grid,),
        in_specs=[row,
                  pl.BlockSpec((D, D), lambda i: (0, 0)),
                  pl.BlockSpec((1, D), lambda i: (0, 0))],
        out_specs=[halfrow, halfrow],
        out_shape=[jax.ShapeDtypeStruct((E, H), f32),
                   jax.ShapeDtypeStruct((E, H), f32)],
    )(e, C_w, C_b)


def _sc_body(ce0, ce1, src, dst, db0, db1, eh, sce, she, znd,
             eo0, eo1, nd,
             acc, srcv, dstv, dscv, cev, eov, dbv, ehv, scv, ssv, shv,
             semL, semG, semW, semE):
    c = lax.axis_index("c")
    s = lax.axis_index("s")
    wid = s * NC + c
    ebase = wid * EPW

    pltpu.sync_copy(sce.at[0], ssv)
    pltpu.sync_copy(she.at[0], shv)

    for half in range(2):
        dbt = db0 if half == 0 else db1
        cet = ce0 if half == 0 else ce1
        eot = eo0 if half == 0 else eo1
        arows = pl.ds(s * NPT, NPT)

        ss = [ssv[pl.ds(half * H + cc * 16, 16)] for cc in range(H // 16)]
        sh = [shv[pl.ds(half * H + cc * 16, 16)] for cc in range(H // 16)]

        pltpu.sync_copy(znd.at[arows], acc.at[arows])
        plsc.subcore_barrier()

        def L_copies(j, b):
            rows = pl.ds(ebase + j * K, K)
            return [(src.at[rows], srcv[b]), (dst.at[rows], dstv[b]),
                    (cet.at[rows], cev[b])]

        def G_copies(b):
            return [(dbt.at[srcv[b]], dbv[b]), (eh.at[dstv[b]], ehv[b])]

        def eout_copy(j):
            return (eov, eot.at[pl.ds(ebase + j * K, K)])

        def issue(copies, sem, **kw):
            for a, o in copies:
                pltpu.async_copy(a, o, sem, **kw)

        def drain(copies, sem):
            for a, o in copies:
                pltpu.make_async_copy(a, o, sem).wait()

        def compute(b):
            def row(i, _):
                for cc in range(H // 16):
                    lo = pl.ds(cc * 16, 16)
                    hi = pl.ds(H + cc * 16, 16)
                    glob = pl.ds(half * H + cc * 16, 16)
                    eij = cev[b][i, lo] + dbv[b][i, lo] + ehv[b][i, glob]
                    eov[i, lo] = jnp.maximum(eij * ss[cc] + sh[cc], 0.0)
                    sig = 1.0 / (1.0 + jnp.exp(-eij))
                    scv[b][i, lo] = sig * dbv[b][i, hi]
                    scv[b][i, hi] = sig
                return 0

            lax.fori_loop(0, K, row, 0)

        def step(j, b, g, eprev):
            drain(G_copies(b), semG[b])

            @pl.when(g >= 1)
            def _():
                pltpu.make_async_copy(scv[b], acc.at[dscv[b]], semW[b]).wait()

            for o in (0, 16, K - 16):
                dscv[b][pl.ds(o, 16)] = dstv[b][pl.ds(o, 16)]

            if eprev is not None:
                guard, jp = eprev
                if guard:
                    @pl.when(g >= 1)
                    def _():
                        a, o = eout_copy(jp)
                        pltpu.make_async_copy(a, o, semE).wait()
                else:
                    a, o = eout_copy(jp)
                    pltpu.make_async_copy(a, o, semE).wait()

            compute(b)
            a, o = eout_copy(j)
            pltpu.async_copy(a, o, semE)
            pltpu.async_copy(scv[b], acc.at[dscv[b]], semW[b], add=True)

            @pl.when(g < NG - 1)
            def _():
                issue(L_copies(j + 2, b), semL[b])

        issue(L_copies(0, 0), semL[0])
        issue(L_copies(1, 1), semL[1])
        drain(L_copies(0, 0), semL[0])
        issue(G_copies(0), semG[0])

        def outer(g, _):
            j0 = 2 * g
            drain(L_copies(j0 + 1, 1), semL[1])
            issue(G_copies(1), semG[1])
            step(j0, 0, g, (True, j0 - 1))
            @pl.when(g < NG - 1)
            def _():
                drain(L_copies(j0 + 2, 0), semL[0])
                issue(G_copies(0), semG[0])
            step(j0 + 1, 1, g, (False, j0))
            return 0

        lax.fori_loop(0, NG, outer, 0)

        a, o = eout_copy(NCH - 1)
        pltpu.make_async_copy(a, o, semE).wait()
        for b in (0, 1):
            pltpu.make_async_copy(scv[b], acc.at[dscv[b]], semW[b]).wait()

        plsc.subcore_barrier()
        pltpu.sync_copy(acc.at[arows], nd.at[c, half, arows])


def _sc_messages(ce0, ce1, src, dst, db0, db1, eh, sce, she):
    f32 = jnp.float32
    i32 = jnp.int32
    znd = jnp.zeros((NP, D), f32)
    mesh = plsc.VectorSubcoreMesh(core_axis_name="c", subcore_axis_name="s",
                                  num_cores=NC, num_subcores=NS)
    two = lambda t: [t, t]
    fn = pl.kernel(
        _sc_body,
        out_type=[
            jax.ShapeDtypeStruct((E, H), f32),
            jax.ShapeDtypeStruct((E, H), f32),
            jax.ShapeDtypeStruct((NC, 2, NP, D), f32),
        ],
        mesh=mesh,
        scratch_types=[
            pltpu.VMEM_SHARED((NP, D), f32),
            two(pltpu.VMEM((K,), i32)),
            two(pltpu.VMEM((K,), i32)),
            two(pltpu.VMEM((K,), i32)),
            two(pltpu.VMEM((K, H), f32)),
            pltpu.VMEM((K, H), f32),
            two(pltpu.VMEM((K, D), f32)),
            two(pltpu.VMEM((K, D), f32)),
            two(pltpu.VMEM((K, D), f32)),
            pltpu.VMEM((D,), f32),
            pltpu.VMEM((D,), f32),
            two(pltpu.SemaphoreType.DMA),
            two(pltpu.SemaphoreType.DMA),
            two(pltpu.SemaphoreType.DMA),
            pltpu.SemaphoreType.DMA,
        ],
    )
    return fn(ce0, ce1, src, dst, db0, db1, eh, sce, she, znd)


def _combine_body(ah, nd, bhw, bhb, bhm, bhv, ho):
    num = jnp.concatenate(
        [nd[0, 0, :, :H] + nd[1, 0, :, :H],
         nd[0, 1, :, :H] + nd[1, 1, :, :H]], axis=1)
    den = jnp.concatenate(
        [nd[0, 0, :, H:] + nd[1, 0, :, H:],
         nd[0, 1, :, H:] + nd[1, 1, :, H:]], axis=1)
    h_new = ah[...] + num / (den + 1e-6)
    scale = bhw[...] / jnp.sqrt(bhv[...] + EPS_BN)
    ho[...] = jnp.maximum(scale * (h_new - bhm[...]) + bhb[...], 0.0)


def _combine_tc(ah, nd, bn_h_w, bn_h_b, bn_h_mean, bn_h_var):
    grid = N // NODE_BLK
    row = pl.BlockSpec((NODE_BLK, D), lambda i: (i, 0))
    v = pl.BlockSpec((1, D), lambda i: (0, 0))
    return pl.pallas_call(
        _combine_body,
        grid=(grid,),
        in_specs=[row,
                  pl.BlockSpec((NC, 2, NODE_BLK, D), lambda i: (0, 0, i, 0)),
                  v, v, v, v],
        out_specs=row,
        out_shape=jax.ShapeDtypeStruct((N, D), jnp.float32),
    )(ah, nd, bn_h_w, bn_h_b, bn_h_mean, bn_h_var)


def kernel(h, e, edge_index, A_w, A_b, B_w, B_b, C_w, C_b, D_w, D_b, E_w, E_b,
           bn_h_w, bn_h_b, bn_h_mean, bn_h_var,
           bn_e_w, bn_e_b, bn_e_mean, bn_e_var):
    r = lambda x: x.reshape(1, D)
    ah, db0, db1, eh, sce, she = _node_tc(
        h, A_w, r(A_b), B_w, r(B_b), D_w, r(D_b), E_w, r(E_b),
        r(bn_e_w), r(bn_e_b), r(bn_e_mean), r(bn_e_var))
    ce0, ce1 = _ce_tc(e, C_w, r(C_b))
    src = edge_index[0]
    dst = edge_index[1]
    eo0, eo1, nd = _sc_messages(ce0, ce1, src, dst, db0, db1, eh, sce, she)
    h_out = _combine_tc(ah, nd, r(bn_h_w), r(bn_h_b),
                        r(bn_h_mean), r(bn_h_var))
    e_out = jnp.concatenate([eo0, eo1], axis=1)
    return (h_out, e_out)
